# baseline (speedup 1.0000x reference)
import jax
import jax.numpy as jnp
from jax import lax
from jax.experimental import pallas as pl
from jax.experimental.pallas import tpu as pltpu
from jax.experimental.pallas import tpu_sc as plsc

_M = 4096
_A = 50
_D = 128
_DP = 131

_NC = 2
_NS = 16
_NW = _NC * _NS

_K = 4
_SM = _M // _K
_MPW = _SM // _NW
_NBUF = 4
_NGROUP = _MPW // _NBUF


def _make_kernel():
    mesh = plsc.VectorSubcoreMesh(core_axis_name="c", subcore_axis_name="s")

    def body(tab_hbm, x_hbm, pos_hbm, out_hbm,
             idx_v, stages, gsems, psems, osems):
        wid = lax.axis_index("s") * _NC + lax.axis_index("c")
        wmol = wid * _MPW

        pltpu.sync_copy(x_hbm.at[pl.ds(wmol, _MPW)], idx_v)

        def issue(b, i):
            g = pltpu.async_copy(
                tab_hbm.at[idx_v.at[i]],
                stages[b].at[:, pl.ds(0, _D)],
                gsems[b],
            )
            p = pltpu.async_copy(
                pos_hbm.at[wmol + i],
                stages[b].at[:, pl.ds(_D, 3)],
                psems[b],
            )
            return g, p

        def write_out(b, i, g, p):
            g.wait()
            p.wait()
            return pltpu.async_copy(
                stages[b], out_hbm.at[wmol + i], osems[b]
            )

        def wait_out(b):
            pltpu.make_async_copy(
                stages[b], out_hbm.at[wmol], osems[b]
            ).wait()

        descs = [issue(b, b) for b in range(_NBUF)]
        for b in range(_NBUF):
            write_out(b, b, *descs[b])

        def grp(g, carry):
            descs = []
            for b in range(_NBUF):
                wait_out(b)
                descs.append(issue(b, g * _NBUF + b))
            for b in range(_NBUF):
                write_out(b, g * _NBUF + b, *descs[b])
            return carry

        lax.fori_loop(1, _NGROUP, grp, 0)

        for b in range(_NBUF):
            wait_out(b)

    return pl.kernel(
        body,
        out_type=jax.ShapeDtypeStruct((_SM, _A, _DP), jnp.float32),
        mesh=mesh,
        scratch_types=[
            pltpu.VMEM((_MPW, _A), jnp.int32),
            [pltpu.VMEM((_A, _DP), jnp.float32) for _ in range(_NBUF)],
            [pltpu.SemaphoreType.DMA for _ in range(_NBUF)],
            [pltpu.SemaphoreType.DMA for _ in range(_NBUF)],
            [pltpu.SemaphoreType.DMA for _ in range(_NBUF)],
        ],
    )


_sc_kernel = _make_kernel()


@jax.jit
def kernel(x, positions, token_emb):
    xi = x.astype(jnp.int32)
    outs = [
        _sc_kernel(
            token_emb,
            lax.slice_in_dim(xi, k * _SM, (k + 1) * _SM, axis=0),
            lax.slice_in_dim(positions, k * _SM, (k + 1) * _SM, axis=0),
        )
        for k in range(_K)
    ]
    return jnp.concatenate(outs, axis=0)


# speedup vs baseline: 1.0889x; 1.0889x over previous
import jax
import jax.numpy as jnp
from jax import lax
from jax.experimental import pallas as pl
from jax.experimental.pallas import tpu as pltpu
from jax.experimental.pallas import tpu_sc as plsc
from jax.experimental.layout import Layout, Format

_M = 4096
_A = 50
_D = 128
_DP = 131

_NC = 2
_NS = 16
_NW = _NC * _NS

_MPW = _M // _NW
_NBUF = 4
_NGROUP = _MPW // _NBUF


def _make_kernel():
    mesh = plsc.VectorSubcoreMesh(core_axis_name="c", subcore_axis_name="s")

    def body(tab_hbm, x_hbm, pos_hbm, out_hbm,
             idx_v, stages, gsems, psems, osems):
        wid = lax.axis_index("s") * _NC + lax.axis_index("c")
        wmol = wid * _MPW

        pltpu.sync_copy(x_hbm.at[pl.ds(wmol, _MPW)], idx_v)

        def issue(b, i):
            g = pltpu.async_copy(
                tab_hbm.at[idx_v.at[i]],
                stages[b].at[:, pl.ds(0, _D)],
                gsems[b],
            )
            p = pltpu.async_copy(
                pos_hbm.at[wmol + i],
                stages[b].at[:, pl.ds(_D, 3)],
                psems[b],
            )
            return g, p

        def write_out(b, i, g, p):
            g.wait()
            p.wait()
            return pltpu.async_copy(
                stages[b], out_hbm.at[wmol + i], osems[b]
            )

        def wait_out(b):
            pltpu.make_async_copy(
                stages[b], out_hbm.at[wmol], osems[b]
            ).wait()

        descs = [issue(b, b) for b in range(_NBUF)]
        for b in range(_NBUF):
            write_out(b, b, *descs[b])

        def grp(g, carry):
            descs = []
            for b in range(_NBUF):
                wait_out(b)
                descs.append(issue(b, g * _NBUF + b))
            for b in range(_NBUF):
                write_out(b, g * _NBUF + b, *descs[b])
            return carry

        lax.fori_loop(1, _NGROUP, grp, 0)

        for b in range(_NBUF):
            wait_out(b)

    return pl.kernel(
        body,
        out_type=jax.ShapeDtypeStruct((_M, _A, _DP), jnp.float32),
        mesh=mesh,
        scratch_types=[
            pltpu.VMEM((_MPW, _A), jnp.int32),
            [pltpu.VMEM((_A, _DP), jnp.float32) for _ in range(_NBUF)],
            [pltpu.SemaphoreType.DMA for _ in range(_NBUF)],
            [pltpu.SemaphoreType.DMA for _ in range(_NBUF)],
            [pltpu.SemaphoreType.DMA for _ in range(_NBUF)],
        ],
    )


_sc_kernel = _make_kernel()


def _kernel_impl(x, positions, token_emb):
    return _sc_kernel(token_emb, x.astype(jnp.int32), positions)


_jitted = None


def kernel(x, positions, token_emb):
    global _jitted
    if _jitted is None:
        fmt = Format(
            Layout(major_to_minor=(0, 1, 2), tiling=()),
            jax.sharding.SingleDeviceSharding(jax.devices()[0]),
        )
        _jitted = jax.jit(_kernel_impl, out_shardings=fmt)
    return _jitted(x, positions, token_emb)


# speedup vs baseline: 1.0893x; 1.0004x over previous
import jax
import jax.numpy as jnp
from jax import lax
from jax.experimental import pallas as pl
from jax.experimental.pallas import tpu as pltpu
from jax.experimental.pallas import tpu_sc as plsc

_M = 4096
_A = 50
_D = 128
_DP = 131

_NC = 2
_NS = 16
_NW = _NC * _NS

_MPW = _M // _NW
_PH = 2
_MPP = _MPW // _PH
_NBUF = 8
_NGROUP = _MPP // _NBUF


def _make_kernel():
    mesh = plsc.VectorSubcoreMesh(core_axis_name="c", subcore_axis_name="s")

    def body(tab_hbm, x_hbm, pos_hbm, out_hbm,
             idx_v, stages, gsems, psems, osems):
        wid = lax.axis_index("s") * _NC + lax.axis_index("c")
        wmol = wid * _MPW

        def issue(b, i):
            g = pltpu.async_copy(
                tab_hbm.at[idx_v.at[i % _MPP]],
                stages[b].at[:, pl.ds(0, _D)],
                gsems[b],
            )
            p = pltpu.async_copy(
                pos_hbm.at[wmol + i],
                stages[b].at[:, pl.ds(_D, 3)],
                psems[b],
            )
            return g, p

        def write_out(b, i, g, p):
            g.wait()
            p.wait()
            return pltpu.async_copy(
                stages[b], out_hbm.at[wmol + i], osems[b]
            )

        def wait_out(b):
            pltpu.make_async_copy(
                stages[b], out_hbm.at[wmol], osems[b]
            ).wait()

        for ph in range(_PH):
            base = ph * _MPP
            pltpu.sync_copy(x_hbm.at[pl.ds(wmol + base, _MPP)], idx_v)

            descs = []
            for b in range(_NBUF):
                if ph > 0:
                    wait_out(b)
                descs.append(issue(b, base + b))
            for b in range(_NBUF):
                write_out(b, base + b, *descs[b])

            def grp(g, carry):
                ds = []
                for b in range(_NBUF):
                    wait_out(b)
                    ds.append(issue(b, base + g * _NBUF + b))
                for b in range(_NBUF):
                    write_out(b, base + g * _NBUF + b, *ds[b])
                return carry

            lax.fori_loop(1, _NGROUP, grp, 0)

        for b in range(_NBUF):
            wait_out(b)

    return pl.kernel(
        body,
        out_type=jax.ShapeDtypeStruct((_M, _A, _DP), jnp.float32),
        mesh=mesh,
        scratch_types=[
            pltpu.VMEM((_MPP, _A), jnp.int32),
            [pltpu.VMEM((_A, _DP), jnp.float32) for _ in range(_NBUF)],
            [pltpu.SemaphoreType.DMA for _ in range(_NBUF)],
            [pltpu.SemaphoreType.DMA for _ in range(_NBUF)],
            [pltpu.SemaphoreType.DMA for _ in range(_NBUF)],
        ],
    )


_sc_kernel = _make_kernel()


@jax.jit
def kernel(x, positions, token_emb):
    return _sc_kernel(token_emb, x.astype(jnp.int32), positions)
